# baseline (speedup 1.0000x reference)
import functools

import jax
import jax.numpy as jnp
from jax import lax
from jax.experimental import pallas as pl
from jax.experimental.pallas import tpu as pltpu
from jax.experimental.pallas import tpu_sc as plsc

_ROWS, _COLS = 128, 100000
_LANES = 16
_NV = _COLS // _LANES
_NW = 32
_ROWS_PER_W = _ROWS // _NW

_mesh = plsc.VectorSubcoreMesh(core_axis_name="c", subcore_axis_name="s")


@functools.partial(
    pl.kernel,
    out_type=jax.ShapeDtypeStruct((_ROWS, _COLS), jnp.float32),
    mesh=_mesh,
    scratch_types=[
        pltpu.VMEM((_COLS,), jnp.float32),
        pltpu.SemaphoreType.DMA,
        pltpu.SemaphoreType.DMA,
    ],
    compiler_params=pltpu.CompilerParams(needs_layout_passes=False),
)
def _sc_log_softmax(x_hbm, o_hbm, buf, sem_in, sem_out):
    wid = lax.axis_index("s") * 2 + lax.axis_index("c")

    for j in range(_ROWS_PER_W):
        row = wid * _ROWS_PER_W + j
        pltpu.async_copy(x_hbm.at[row], buf, sem_in).wait()

        def max_body(i, m16):
            return jnp.maximum(m16, buf[pl.ds(i * _LANES, _LANES)])

        m16 = lax.fori_loop(
            0, _NV, max_body, jnp.full((_LANES,), -jnp.inf, jnp.float32),
            unroll=5,
        )
        m = jnp.max(m16)
        mb = jnp.full((_LANES,), m, jnp.float32)

        def sum_body(i, s16):
            return s16 + jnp.exp(buf[pl.ds(i * _LANES, _LANES)] - mb)

        s16 = lax.fori_loop(
            0, _NV, sum_body, jnp.zeros((_LANES,), jnp.float32), unroll=5
        )
        s = jnp.sum(s16)

        sv = jnp.full((_LANES,), s, jnp.float32)
        bits = plsc.bitcast(sv, jnp.int32)
        y = (bits.astype(jnp.float32) * (1.0 / 8388608.0)
             - 126.95699) * 0.6931471805599453
        for _ in range(4):
            y = y + sv * jnp.exp(-y) - 1.0
        lse = y + mb

        def sub_body(i, carry):
            sl = pl.ds(i * _LANES, _LANES)
            buf[sl] = buf[sl] - lse
            return carry

        lax.fori_loop(0, _NV, sub_body, 0, unroll=5)
        pltpu.async_copy(buf, o_hbm.at[row], sem_out).wait()


def kernel(logits):
    return _sc_log_softmax(logits)
